# baseline (speedup 1.0000x reference)
import jax
import jax.numpy as jnp
from jax import lax
from jax.experimental import pallas as pl
from jax.experimental.pallas import tpu as pltpu
from jax.experimental.pallas import tpu_sc as plsc

N = 10000
E = 160000
D = 256
DC = 128
NTILES = 16
ET = E // NTILES
B = 64
ETP = 10240
NB = ETP // B
SBN = 10
NSB = NB // SBN
NBUF = 4
NP = 10240
RT = NP // NTILES


def _sc_segment_kernel(xh, src3, dst3, w3, ones1):
    mesh = plsc.VectorSubcoreMesh(core_axis_name="c", subcore_axis_name="s")

    def body(xh_hbm, src_hbm, dst_hbm, w_hbm, ones_hbm, agg_out, deg_out,
             src_v, dst_v, w_v, msg_a, msg_b, msg_c, msg_d, ones_v,
             sem_ga, sem_gb, sem_gc, sem_gd, sem_sa, sem_sb, sem_sc,
             sem_sd2, sem_d, acc_s, deg_s):
        cid = lax.axis_index("c")
        tid = lax.axis_index("s")
        row0 = tid * RT
        msgs = (msg_a, msg_b, msg_c, msg_d)
        gsems = (sem_ga, sem_gb, sem_gc, sem_gd)
        ssems = (sem_sa, sem_sb, sem_sc, sem_sd2)

        pltpu.sync_copy(ones_hbm, ones_v)

        def zero_msg_a(r, _):
            for q in range(DC // 16):
                msg_a[r, pl.ds(q * 16, 16)] = jnp.zeros((16,), jnp.float32)
            return 0

        def scale(buf, j):
            def group_body(g, _):
                wv = w_v[j, pl.ds(g * 16, 16)]
                for k in range(16):
                    we = wv[k]
                    for q in range(DC // 16):
                        sl = pl.ds(q * 16, 16)
                        buf[g * 16 + k, sl] = buf[g * 16 + k, sl] * we
                return 0

            lax.fori_loop(0, B // 16, group_body, 0)

        def do_pass(p, _):
            with_deg = jnp.logical_and(cid == 0, p == 0)
            chunk = 2 * cid + p
            off = chunk * NP

            lax.fori_loop(0, B, zero_msg_a, 0)
            for i in range(RT // B):
                pltpu.sync_copy(msg_a, acc_s.at[pl.ds(row0 + i * B, B)])

            @pl.when(with_deg)
            def _():
                for i in range(RT // DC):
                    pltpu.sync_copy(msg_a.at[0],
                                    deg_s.at[pl.ds(row0 + i * DC, DC)])

            plsc.subcore_barrier()

            def super_body(sb, _):
                pltpu.sync_copy(src_hbm.at[tid, sb], src_v)
                pltpu.sync_copy(dst_hbm.at[tid, sb], dst_v)
                pltpu.sync_copy(w_hbm.at[tid, sb], w_v)

                def add_off(r, _):
                    for q in range(B // 16):
                        sl = pl.ds(q * 16, 16)
                        src_v[r, sl] = src_v[r, sl] + off
                    return 0

                lax.fori_loop(0, SBN, add_off, 0)

                gd = [None] * SBN
                sd = [None] * SBN

                def gather(j):
                    return pltpu.async_copy(xh_hbm.at[src_v.at[j]],
                                            msgs[j % NBUF],
                                            gsems[j % NBUF])

                gd[0] = gather(0)
                gd[1] = gather(1)
                for j in range(SBN):
                    if j >= 2:
                        sd[j - 2].wait()
                    if j + 2 < SBN:
                        gd[j + 2] = gather(j + 2)
                    gd[j].wait()
                    buf = msgs[j % NBUF]
                    scale(buf, j)
                    sd[j] = pltpu.async_copy(buf, acc_s.at[dst_v.at[j]],
                                             ssems[j % NBUF], add=True)

                    @pl.when(with_deg)
                    def _(jj=j):
                        pltpu.async_copy(ones_v, deg_s.at[dst_v.at[jj]],
                                         sem_d, add=True)

                sd[SBN - 2].wait()
                sd[SBN - 1].wait()

                @pl.when(with_deg)
                def _():
                    for jj in range(SBN):
                        pltpu.make_async_copy(
                            ones_v, deg_s.at[dst_v.at[jj]], sem_d).wait()
                return 0

            lax.fori_loop(0, NSB, super_body, 0)
            plsc.subcore_barrier()

            for i in range(RT // B):
                pltpu.sync_copy(acc_s.at[pl.ds(row0 + i * B, B)], msg_a)
                pltpu.sync_copy(msg_a,
                                agg_out.at[chunk, pl.ds(row0 + i * B, B)])

            @pl.when(with_deg)
            def _():
                for i in range(RT // DC):
                    pltpu.sync_copy(deg_s.at[pl.ds(row0 + i * DC, DC)],
                                    msg_a.at[0])
                    pltpu.sync_copy(msg_a.at[0],
                                    deg_out.at[pl.ds(row0 + i * DC, DC)])
            return 0

        lax.fori_loop(0, 2, do_pass, 0)

    f = pl.kernel(
        body,
        out_type=(jax.ShapeDtypeStruct((4, NP, DC), jnp.float32),
                  jax.ShapeDtypeStruct((NP,), jnp.float32)),
        mesh=mesh,
        scratch_types=[
            pltpu.VMEM((SBN, B), jnp.int32),
            pltpu.VMEM((SBN, B), jnp.int32),
            pltpu.VMEM((SBN, B), jnp.float32),
            pltpu.VMEM((B, DC), jnp.float32),
            pltpu.VMEM((B, DC), jnp.float32),
            pltpu.VMEM((B, DC), jnp.float32),
            pltpu.VMEM((B, DC), jnp.float32),
            pltpu.VMEM((B,), jnp.float32),
            pltpu.SemaphoreType.DMA,
            pltpu.SemaphoreType.DMA,
            pltpu.SemaphoreType.DMA,
            pltpu.SemaphoreType.DMA,
            pltpu.SemaphoreType.DMA,
            pltpu.SemaphoreType.DMA,
            pltpu.SemaphoreType.DMA,
            pltpu.SemaphoreType.DMA,
            pltpu.SemaphoreType.DMA,
            pltpu.VMEM_SHARED((NP, DC), jnp.float32),
            pltpu.VMEM_SHARED((NP,), jnp.float32),
        ],
    )
    return f(xh, src3, dst3, w3, ones1)


def _tc_body(a0, a1, a2, a3, deg1, x, h, win, wis, whn, whs, bi, bh, out):
    deg = jnp.maximum(deg1[...], 1.0)
    m0 = a0[...] / deg
    m1 = a1[...] / deg
    m2 = a2[...] / deg
    m3 = a3[...] / deg
    hp = lax.Precision.HIGHEST
    acc = jnp.dot(x[...], wis[...], precision=hp)
    acc += jnp.dot(h[...], whs[...], precision=hp)
    acc += jnp.dot(m0, win[0:DC, :], precision=hp)
    acc += jnp.dot(m1, win[DC:D, :], precision=hp)
    acc += jnp.dot(m2, whn[0:DC, :], precision=hp)
    acc += jnp.dot(m3, whn[DC:D, :], precision=hp)
    out[...] = jnp.tanh(acc + bi[...] + bh[...])


def _tc_dense(a0, a1, a2, a3, deg1, X, H, Wi_neigh, Wi_self, Wh_neigh,
              Wh_self, bi, bh):
    R = 400
    grid = (N // R,)

    def row_spec(w):
        return pl.BlockSpec((R, w), lambda i: (i, 0))

    def full_spec(r, c):
        return pl.BlockSpec((r, c), lambda i: (0, 0))

    return pl.pallas_call(
        _tc_body,
        grid=grid,
        in_specs=[
            row_spec(DC), row_spec(DC), row_spec(DC), row_spec(DC),
            pl.BlockSpec((R, 1), lambda i: (i, 0)),
            row_spec(D), row_spec(D),
            full_spec(D, D), full_spec(D, D), full_spec(D, D),
            full_spec(D, D), full_spec(1, D), full_spec(1, D),
        ],
        out_specs=row_spec(D),
        out_shape=jax.ShapeDtypeStruct((N, D), jnp.float32),
    )(a0, a1, a2, a3, deg1, X, H, Wi_neigh, Wi_self, Wh_neigh, Wh_self,
      bi.reshape(1, D), bh.reshape(1, D))


def kernel(X, edge_index, edge_weight, H, Wi_neigh, Wi_self, bi, Wh_neigh,
           Wh_self, bh):
    pad = ETP - ET
    src = jnp.pad(edge_index[0].reshape(NTILES, ET), ((0, 0), (0, pad)),
                  constant_values=0).reshape(NTILES, NSB, SBN, B)
    dst = jnp.pad(edge_index[1].reshape(NTILES, ET), ((0, 0), (0, pad)),
                  constant_values=NP - 1).reshape(NTILES, NSB, SBN, B)
    w = jnp.pad(edge_weight.reshape(NTILES, ET), ((0, 0), (0, pad)),
                constant_values=0.0).reshape(NTILES, NSB, SBN, B)
    Xp = jnp.pad(X, ((0, NP - N), (0, 0)))
    Hp = jnp.pad(H, ((0, NP - N), (0, 0)))
    xh = jnp.concatenate([Xp[:, :DC], Xp[:, DC:], Hp[:, :DC], Hp[:, DC:]],
                         axis=0)
    ones1 = jnp.ones((B,), jnp.float32)

    agg, deg = _sc_segment_kernel(xh, src, dst, w, ones1)
    agg = agg[:, :N]
    deg1 = deg[:N].reshape(N, 1)
    return _tc_dense(agg[0], agg[1], agg[2], agg[3], deg1, X, H, Wi_neigh,
                     Wi_self, Wh_neigh, Wh_self, bi, bh)


# speedup vs baseline: 1.0435x; 1.0435x over previous
import jax
import jax.numpy as jnp
from jax import lax
from jax.experimental import pallas as pl
from jax.experimental.pallas import tpu as pltpu
from jax.experimental.pallas import tpu_sc as plsc

N = 10000
E = 160000
D = 256
DC = 128
NTILES = 16
ET = E // NTILES
B = 64
ETP = 10240
NB = ETP // B
SBN = 10
NSB = NB // SBN
NBUF = 4
NP = 10240
RT = NP // NTILES
NBLK = RT // B


def _sc_segment_kernel(xh, edata, w4, ones1):
    mesh = plsc.VectorSubcoreMesh(core_axis_name="c", subcore_axis_name="s")

    def body(xh_hbm, ed_hbm, w_hbm, ones_hbm, agg_out, deg_out,
             ed_v, w_v, msg_a, msg_b, msg_c, msg_d, zero_v, ones_v,
             sem_ga, sem_gb, sem_gc, sem_gd, sem_sa, sem_sb, sem_sc,
             sem_sd2, sem_d, acc_s, deg_s):
        cid = lax.axis_index("c")
        tid = lax.axis_index("s")
        row0 = tid * RT
        msgs = (msg_a, msg_b, msg_c, msg_d)
        gsems = (sem_ga, sem_gb, sem_gc, sem_gd)
        ssems = (sem_sa, sem_sb, sem_sc, sem_sd2)
        core0 = cid == 0

        pltpu.sync_copy(ones_hbm, ones_v)

        def fill_zero(r, _):
            for q in range(DC // 16):
                zero_v[r, pl.ds(q * 16, 16)] = jnp.zeros((16,), jnp.float32)
            return 0

        lax.fori_loop(0, B, fill_zero, 0)

        zinit = [pltpu.async_copy(zero_v, acc_s.at[pl.ds(row0 + i * B, B)],
                                  sem_d) for i in range(NBLK)]

        @pl.when(core0)
        def _():
            for i in range(RT // DC):
                pltpu.sync_copy(zero_v.at[0],
                                deg_s.at[pl.ds(row0 + i * DC, DC)])

        for z in zinit:
            z.wait()
        plsc.subcore_barrier()

        def scale(buf, j):
            def group_body(g, _):
                wv = w_v[j, pl.ds(g * 16, 16)]
                for k in range(16):
                    we = wv[k]
                    for q in range(DC // 16):
                        sl = pl.ds(q * 16, 16)
                        buf[g * 16 + k, sl] = buf[g * 16 + k, sl] * we
                return 0

            lax.fori_loop(0, B // 16, group_body, 0)

        def do_pass(p, _):
            with_deg = jnp.logical_and(core0, p == 0)
            chunk = 2 * cid + p
            off = chunk * NP

            def super_body(sb, _):
                pltpu.sync_copy(ed_hbm.at[tid, sb], ed_v)
                pltpu.sync_copy(w_hbm.at[tid, sb], w_v)

                def add_off(r, _):
                    for q in range(B // 16):
                        sl = pl.ds(q * 16, 16)
                        ed_v[0, r, sl] = ed_v[0, r, sl] + off
                    return 0

                lax.fori_loop(0, SBN, add_off, 0)

                gd = [None] * SBN
                sd = [None] * SBN

                def gather(j):
                    return pltpu.async_copy(xh_hbm.at[ed_v.at[0, j]],
                                            msgs[j % NBUF],
                                            gsems[j % NBUF])

                gd[0] = gather(0)
                gd[1] = gather(1)
                for j in range(SBN):
                    if j >= 2:
                        sd[j - 2].wait()
                    if j + 2 < SBN:
                        gd[j + 2] = gather(j + 2)
                    gd[j].wait()
                    buf = msgs[j % NBUF]
                    scale(buf, j)
                    sd[j] = pltpu.async_copy(buf,
                                             acc_s.at[ed_v.at[1, j]],
                                             ssems[j % NBUF], add=True)

                    @pl.when(with_deg)
                    def _(jj=j):
                        pltpu.async_copy(ones_v, deg_s.at[ed_v.at[1, jj]],
                                         sem_d, add=True)

                sd[SBN - 2].wait()
                sd[SBN - 1].wait()

                @pl.when(with_deg)
                def _():
                    for jj in range(SBN):
                        pltpu.make_async_copy(
                            ones_v, deg_s.at[ed_v.at[1, jj]], sem_d).wait()
                return 0

            lax.fori_loop(0, NSB, super_body, 0)
            plsc.subcore_barrier()

            def ablk(i):
                return acc_s.at[pl.ds(row0 + i * B, B)]

            rd = [None] * NBLK
            wr = [None] * NBLK
            zd = [None] * NBLK
            rd[0] = pltpu.async_copy(ablk(0), msg_a, gsems[0])
            for i in range(NBLK):
                fbuf = msgs[i % 2]
                rd[i].wait()
                zd[i] = pltpu.async_copy(zero_v, ablk(i), sem_d)
                if i >= 1:
                    wr[i - 1].wait()
                if i + 1 < NBLK:
                    rd[i + 1] = pltpu.async_copy(ablk(i + 1),
                                                 msgs[(i + 1) % 2],
                                                 gsems[(i + 1) % 2])
                wr[i] = pltpu.async_copy(
                    fbuf, agg_out.at[chunk, pl.ds(row0 + i * B, B)],
                    ssems[i % 2])
            wr[NBLK - 1].wait()
            for z in zd:
                z.wait()

            @pl.when(with_deg)
            def _():
                for i in range(RT // DC):
                    pltpu.sync_copy(deg_s.at[pl.ds(row0 + i * DC, DC)],
                                    msg_a.at[0])
                    pltpu.sync_copy(msg_a.at[0],
                                    deg_out.at[pl.ds(row0 + i * DC, DC)])
                    pltpu.sync_copy(zero_v.at[0],
                                    deg_s.at[pl.ds(row0 + i * DC, DC)])
            plsc.subcore_barrier()
            return 0

        lax.fori_loop(0, 2, do_pass, 0)

    f = pl.kernel(
        body,
        out_type=(jax.ShapeDtypeStruct((4, NP, DC), jnp.float32),
                  jax.ShapeDtypeStruct((NP,), jnp.float32)),
        mesh=mesh,
        scratch_types=[
            pltpu.VMEM((2, SBN, B), jnp.int32),
            pltpu.VMEM((SBN, B), jnp.float32),
            pltpu.VMEM((B, DC), jnp.float32),
            pltpu.VMEM((B, DC), jnp.float32),
            pltpu.VMEM((B, DC), jnp.float32),
            pltpu.VMEM((B, DC), jnp.float32),
            pltpu.VMEM((B, DC), jnp.float32),
            pltpu.VMEM((B,), jnp.float32),
            pltpu.SemaphoreType.DMA,
            pltpu.SemaphoreType.DMA,
            pltpu.SemaphoreType.DMA,
            pltpu.SemaphoreType.DMA,
            pltpu.SemaphoreType.DMA,
            pltpu.SemaphoreType.DMA,
            pltpu.SemaphoreType.DMA,
            pltpu.SemaphoreType.DMA,
            pltpu.SemaphoreType.DMA,
            pltpu.VMEM_SHARED((NP, DC), jnp.float32),
            pltpu.VMEM_SHARED((NP,), jnp.float32),
        ],
    )
    return f(xh, edata, w4, ones1)


def _tc_body(a4, deg1, x, h, win, wis, whn, whs, bi, bh, out):
    deg = jnp.maximum(deg1[...], 1.0)
    hp = lax.Precision.HIGHEST
    acc = jnp.dot(x[...], wis[...], precision=hp)
    acc += jnp.dot(h[...], whs[...], precision=hp)
    acc += jnp.dot(a4[0] / deg, win[0:DC, :], precision=hp)
    acc += jnp.dot(a4[1] / deg, win[DC:D, :], precision=hp)
    acc += jnp.dot(a4[2] / deg, whn[0:DC, :], precision=hp)
    acc += jnp.dot(a4[3] / deg, whn[DC:D, :], precision=hp)
    out[...] = jnp.tanh(acc + bi[...] + bh[...])


def _tc_dense(a4, deg1, X, H, Wi_neigh, Wi_self, Wh_neigh, Wh_self, bi, bh):
    R = 400
    grid = (N // R,)

    def row_spec(w):
        return pl.BlockSpec((R, w), lambda i: (i, 0))

    def full_spec(r, c):
        return pl.BlockSpec((r, c), lambda i: (0, 0))

    return pl.pallas_call(
        _tc_body,
        grid=grid,
        in_specs=[
            pl.BlockSpec((4, R, DC), lambda i: (0, i, 0)),
            pl.BlockSpec((R, 1), lambda i: (i, 0)),
            row_spec(D), row_spec(D),
            full_spec(D, D), full_spec(D, D), full_spec(D, D),
            full_spec(D, D), full_spec(1, D), full_spec(1, D),
        ],
        out_specs=row_spec(D),
        out_shape=jax.ShapeDtypeStruct((N, D), jnp.float32),
    )(a4, deg1, X, H, Wi_neigh, Wi_self, Wh_neigh, Wh_self,
      bi.reshape(1, D), bh.reshape(1, D))


def kernel(X, edge_index, edge_weight, H, Wi_neigh, Wi_self, bi, Wh_neigh,
           Wh_self, bh):
    pad = ETP - ET
    src = jnp.pad(edge_index[0].reshape(NTILES, ET), ((0, 0), (0, pad)),
                  constant_values=0).reshape(NTILES, NSB, SBN, B)
    dst = jnp.pad(edge_index[1].reshape(NTILES, ET), ((0, 0), (0, pad)),
                  constant_values=NP - 1).reshape(NTILES, NSB, SBN, B)
    w = jnp.pad(edge_weight.reshape(NTILES, ET), ((0, 0), (0, pad)),
                constant_values=0.0).reshape(NTILES, NSB, SBN, B)
    edata = jnp.stack([src, dst], axis=2)
    Xp = jnp.pad(X, ((0, NP - N), (0, 0)))
    Hp = jnp.pad(H, ((0, NP - N), (0, 0)))
    xh = jnp.concatenate([Xp[:, :DC], Xp[:, DC:], Hp[:, :DC], Hp[:, DC:]],
                         axis=0)
    ones1 = jnp.ones((B,), jnp.float32)

    agg, deg = _sc_segment_kernel(xh, edata, w, ones1)
    deg1 = deg.reshape(NP, 1)
    return _tc_dense(agg, deg1, X, H, Wi_neigh, Wi_self, Wh_neigh, Wh_self,
                     bi, bh)
